# Initial kernel scaffold; baseline (speedup 1.0000x reference)
import jax
import jax.numpy as jnp
from jax.experimental import pallas as pl


def kernel(boxes, attention, Ws):
    raise NotImplementedError("write your pallas kernel here")


# speedup vs baseline: 6.3846x; 6.3846x over previous
import functools

import jax
import jax.numpy as jnp
from jax.experimental import pallas as pl
from jax.experimental.pallas import tpu as pltpu

_NP = 10
_N = 100
_D = 16
_BLK = 128


def _pw_multi(xp, coef_ref, rows):
    res = [coef_ref[r, 0] * (xp + 1.0) for r in rows]
    for s in range(_D):
        t = xp if s == 0 else jnp.maximum(xp - float(s), 0.0)
        for j, r in enumerate(rows):
            res[j] = res[j] + coef_ref[r, s + 1] * t
    return res


def _block_kernel(att_ref, boxes_ref, coef_ref, fac_ref, out_ref):
    att_all = att_ref[...]
    iota_n = jax.lax.broadcasted_iota(jnp.int32, (_N, _BLK), 0)

    work = att_all
    vals, coords = [], [[] for _ in range(4)]
    for _ in range(_NP):
        m = jnp.max(work, axis=0, keepdims=True)
        cand = jnp.where(work == m, iota_n, _N)
        sel = jnp.min(cand, axis=0, keepdims=True)
        onehot = iota_n == sel
        vals.append(m)
        ohf = onehot.astype(jnp.float32)
        for c in range(4):
            coords[c].append(
                jnp.sum(boxes_ref[c] * ohf, axis=0, keepdims=True))
        work = jnp.where(onehot, -jnp.inf, work)

    att_top = jnp.concatenate(vals, axis=0)
    y0, x0, y1, x1 = (jnp.concatenate(cs, axis=0) for cs in coords)

    e = jnp.exp(-jnp.abs(att_top))
    att = jnp.where(att_top >= 0.0, 1.0 / (1.0 + e), e / (1.0 + e))

    ai = att[:, None, :]
    aj = att[None, :, :]
    A = ai * aj

    h = jnp.maximum(y1 - y0, 0.0)
    w = jnp.maximum(x1 - x0, 0.0)
    areas = h * w
    mny = jnp.maximum(y0[:, None, :], y0[None, :, :])
    mnx = jnp.maximum(x0[:, None, :], x0[None, :, :])
    mxy = jnp.minimum(y1[:, None, :], y1[None, :, :])
    mxx = jnp.minimum(x1[:, None, :], x1[None, :, :])
    ia = jnp.maximum(mxy - mny, 0.0) * jnp.maximum(mxx - mnx, 0.0)
    iou = ia / (areas[:, None, :] + areas[None, :, :] - ia + 1e-12)
    Dm = 1.0 - iou

    (plA0,) = _pw_multi(A * 16.0, coef_ref, [0])
    plD1, plD6 = _pw_multi(Dm * 16.0, coef_ref, [1, 6])
    att_diff = jnp.abs(ai - aj)
    (sim,) = _pw_multi((1.0 - att_diff) * 16.0, coef_ref, [2])

    A_tilde = plA0 * plD1
    s_i = fac_ref[0, 0] * jnp.sum(sim, axis=1)
    score = A_tilde / (s_i[:, None, :] * s_i[None, :, :])
    (corr_num,) = _pw_multi(att * att * 16.0, coef_ref, [0])
    corr = corr_num / s_i
    mod_E = (jnp.sum(jnp.sum(score, axis=1), axis=0, keepdims=True)
             + jnp.sum(corr, axis=0, keepdims=True))
    c = jnp.sqrt(mod_E + 1e-20)

    cc = jnp.clip(c, 0.0, float(_NP))
    ip = cc.astype(jnp.int32)
    fp = cc - jnp.trunc(cc)
    iota11 = jax.lax.broadcasted_iota(jnp.int32, (_NP + 1, _BLK), 0)
    left = (iota11 == ip).astype(jnp.float32)
    right = (iota11 == jnp.minimum(ip + 1, _NP)).astype(jnp.float32)
    o = (1.0 - fp) * left + fp * right

    (pl5,) = _pw_multi(att * 16.0, coef_ref, [5])
    p_a = jnp.abs(pl5 - 0.5)
    pam = jnp.sum(p_a, axis=0, keepdims=True) / float(_NP)
    p_d = jnp.abs(plD6 - 0.5)
    pdm = jnp.sum(jnp.sum(p_d, axis=1) / float(_NP),
                  axis=0, keepdims=True) / float(_NP)
    (gate,) = _pw_multi((pam + pdm) * 16.0, coef_ref, [7])
    out_ref[...] = gate * o


@jax.jit
def kernel(boxes, attention, Ws):
    B = attention.shape[0]
    att_t = attention.T
    boxes_t = boxes.transpose(1, 2, 0)

    aw = jnp.abs(Ws)
    nw = aw / jnp.sum(aw, axis=1, keepdims=True)
    coef = jnp.concatenate([nw[:, :1], nw[:, 1:] - nw[:, :-1]], axis=1)
    cs2_top = jnp.cumsum(nw[2])[-1]
    fac = (cs2_top ** _NP).reshape(1, 1)

    out_t = pl.pallas_call(
        _block_kernel,
        grid=(B // _BLK,),
        in_specs=[
            pl.BlockSpec((_N, _BLK), lambda i: (0, i)),
            pl.BlockSpec((4, _N, _BLK), lambda i: (0, 0, i)),
            pl.BlockSpec(memory_space=pltpu.SMEM),
            pl.BlockSpec(memory_space=pltpu.SMEM),
        ],
        out_specs=pl.BlockSpec((_NP + 1, _BLK), lambda i: (0, i)),
        out_shape=jax.ShapeDtypeStruct((_NP + 1, B), jnp.float32),
    )(att_t, boxes_t, coef, fac)
    return out_t.T
